# baseline (speedup 1.0000x reference)
import functools
import math

import jax
import jax.numpy as jnp
from jax.experimental import pallas as pl
from jax.experimental.pallas import tpu as pltpu

_BB = 8


def _body(onehot_ref, mem_ref, emb_ref, qWT_ref, qb_ref, oWT_ref, ob_ref,
          out_ref):
    D = emb_ref.shape[1]
    e = jnp.dot(onehot_ref[...], emb_ref[...],
                preferred_element_type=jnp.float32)
    q = jnp.dot(e, qWT_ref[...], preferred_element_type=jnp.float32)
    q = q + qb_ref[...]
    q = q * (1.0 / math.sqrt(D))

    sims_rows = []
    for b in range(_BB):
        mem_b = mem_ref[b]
        s_b = jax.lax.dot_general(
            q[b:b + 1], mem_b,
            dimension_numbers=(((1,), (1,)), ((), ())),
            preferred_element_type=jnp.float32)
        sims_rows.append(s_b)
    sims = jnp.concatenate(sims_rows, axis=0)

    m = jnp.max(sims, axis=-1, keepdims=True)
    p = jnp.exp(sims - m)
    s = jnp.sum(p, axis=-1, keepdims=True)

    pooled_rows = []
    for b in range(_BB):
        mem_b = mem_ref[b]
        pb = jax.lax.dot_general(
            p[b:b + 1], mem_b,
            dimension_numbers=(((1,), (0,)), ((), ())),
            preferred_element_type=jnp.float32)
        pooled_rows.append(pb)
    pooled = jnp.concatenate(pooled_rows, axis=0)
    pooled = pooled * (1.0 / s)

    out_ref[...] = jnp.dot(pooled, oWT_ref[...],
                           preferred_element_type=jnp.float32) + ob_ref[...]


def kernel(query, memory, emb, qW, qb, oW, ob):
    B, M, D = memory.shape
    V = oW.shape[0]
    onehot = jax.nn.one_hot(query, emb.shape[0], dtype=jnp.float32)
    qWT = qW.T
    oWT = oW.T
    qb2 = qb.reshape(1, D)
    ob2 = ob.reshape(1, V)

    grid = (B // _BB,)
    out = pl.pallas_call(
        _body,
        out_shape=jax.ShapeDtypeStruct((B, V), jnp.float32),
        grid=grid,
        in_specs=[
            pl.BlockSpec((_BB, emb.shape[0]), lambda i: (i, 0)),
            pl.BlockSpec((_BB, M, D), lambda i: (i, 0, 0)),
            pl.BlockSpec(emb.shape, lambda i: (0, 0)),
            pl.BlockSpec((D, D), lambda i: (0, 0)),
            pl.BlockSpec((1, D), lambda i: (0, 0)),
            pl.BlockSpec((D, V), lambda i: (0, 0)),
            pl.BlockSpec((1, V), lambda i: (0, 0)),
        ],
        out_specs=pl.BlockSpec((_BB, V), lambda i: (i, 0)),
        compiler_params=pltpu.CompilerParams(
            dimension_semantics=("arbitrary",),
            vmem_limit_bytes=48 * 1024 * 1024,
        ),
        name="read_gate_fused",
    )(onehot, memory, emb, qWT, qb2, oWT, ob2)
    return out
